# baseline (speedup 1.0000x reference)
import functools

import jax
import jax.numpy as jnp
from jax import lax
from jax.experimental import pallas as pl
from jax.experimental.pallas import tpu as pltpu
from jax.experimental.pallas import tpu_sc as plsc

_K = 16
_EPS = 1e-5
_R = 256

_NC = 2
_NS = 16
_NW = _NC * _NS


def _topk_idx_kernel(xr_ref, xc_ref, w2_ref, idx_ref, *, n, npts):
    bi = pl.program_id(0)
    xrb = xr_ref[0]
    xcb = xc_ref[0]
    g = jax.lax.dot_general(xrb, xcb, (((0,), (0,)), ((), ())),
                            preferred_element_type=jnp.float32)
    inner = -2.0 * g
    xx_r = jnp.sum(xrb * xrb, axis=0)
    xx_c = jnp.sum(xcb * xcb, axis=0)
    pd = (-xx_r[:, None] - inner) - xx_c[None, :]

    iota = jax.lax.broadcasted_iota(jnp.int32, pd.shape, 1)
    r = pd.shape[0]
    ng = n // 16
    giota = jax.lax.broadcasted_iota(jnp.int32, (r, ng), 1)
    w2 = w2_ref[...]
    base = bi * n
    cols = []
    for _ in range(_K):
        m = jnp.max(pd, axis=1)
        eqf = (pd == m[:, None]).astype(jnp.float32)
        s = jax.lax.dot_general(eqf, w2, (((1,), (0,)), ((), ())),
                                preferred_element_type=jnp.float32)
        gstar = jnp.min(jnp.where(s > 0.0, giota, ng), axis=1)
        firsthot = giota == gstar[:, None]
        sval = jnp.max(jnp.where(firsthot, s, 0.0), axis=1)
        lstar = 127 - (jax.lax.bitcast_convert_type(sval, jnp.int32) >> 23)
        jmin = gstar * 16 + lstar
        onehot_b = iota == jmin[:, None]
        pd = jnp.where(onehot_b, -jnp.inf, pd)
        cols.append((jmin + base)[:, None])
    idx_ref[...] = jnp.concatenate(cols, axis=1)


def _sc_gather_max_kernel(xt_hbm, idx_hbm, out_hbm, idxv, rows0, rows1, acc,
                          sem0, sem1, *, pts_per_w, c):
    wid = lax.axis_index("s") * _NC + lax.axis_index("c")
    pltpu.sync_copy(idx_hbm.at[wid], idxv)
    nchunks = idxv.shape[0]
    ppc = 128 // _K
    ncc = c // 16

    def _fire(ci, rbuf, sem):
        pltpu.make_async_copy(xt_hbm.at[idxv.at[ci]], rbuf, sem).start()

    def _drain(rbuf, sem):
        pltpu.make_async_copy(xt_hbm.at[idxv.at[0]], rbuf, sem).wait()

    def _compute(ci, rbuf):
        def pt_body(p, carry):
            orow = ci * ppc + p
            for cc in range(ncc):
                m = rbuf[p * _K, pl.ds(cc * 16, 16)]
                for j in range(1, _K):
                    m = jnp.maximum(m, rbuf[p * _K + j, pl.ds(cc * 16, 16)])
                acc[orow, pl.ds(cc * 16, 16)] = m
            return carry
        lax.fori_loop(0, ppc, pt_body, 0, unroll=True)

    _fire(0, rows0, sem0)

    def body2(i2, carry):
        ci0 = i2 * 2
        _fire(ci0 + 1, rows1, sem1)
        _drain(rows0, sem0)
        _compute(ci0, rows0)

        @pl.when(i2 < nchunks // 2 - 1)
        def _():
            _fire(ci0 + 2, rows0, sem0)

        _drain(rows1, sem1)
        _compute(ci0 + 1, rows1)
        return carry

    lax.fori_loop(0, nchunks // 2, body2, 0)
    pltpu.sync_copy(acc, out_hbm.at[pl.ds(wid * pts_per_w, pts_per_w)])


def _conv_stats_kernel(xa_ref, w_ref, b_ref, y_ref, maug_ref):
    bi = pl.program_id(0)
    nb = pl.program_id(1)
    xa = xa_ref[...]
    y = jax.lax.dot_general(w_ref[...], xa, (((1,), (1,)), ((), ())),
                            preferred_element_type=jnp.float32)
    y = y + b_ref[...]
    y_ref[0] = y
    r = y.shape[1]
    aug = jnp.concatenate([y, jnp.ones((1, r), jnp.float32)], axis=0)
    contrib = jax.lax.dot_general(aug, aug, (((1,), (1,)), ((), ())),
                                  preferred_element_type=jnp.float32,
                                  precision=jax.lax.Precision.HIGHEST)

    @pl.when(jnp.logical_and(bi == 0, nb == 0))
    def _():
        maug_ref[...] = jnp.zeros_like(maug_ref)

    maug_ref[...] += contrib


def _bn_kernel(y_ref, maug_ref, gamma_ref, beta_ref, out_ref, *, cnt, c):
    maug = maug_ref[...]
    m = maug[:c, :c]
    s1 = maug[c, :c]
    eye = (jax.lax.broadcasted_iota(jnp.int32, (c, c), 0)
           == jax.lax.broadcasted_iota(jnp.int32, (c, c), 1))
    diag = jnp.sum(jnp.where(eye, m, 0.0), axis=1)
    mean = s1 / cnt
    var = diag / cnt - mean * mean
    inv = 1.0 / jnp.sqrt(var + _EPS)
    scale = gamma_ref[...][:, 0] * inv
    shift = beta_ref[...][:, 0] - mean * scale
    yb = y_ref[0]
    out_ref[0] = jnp.maximum(yb * scale[:, None] + shift[:, None], 0.0)


def kernel(x, conv_w, conv_b, bn_gamma, bn_beta):
    b, c, n = x.shape
    r = _R
    nb = n // r
    npts = b * n
    bias = conv_b.reshape(c, 1)
    gamma = bn_gamma.reshape(c, 1)
    beta = bn_beta.reshape(c, 1)

    ng = n // 16
    jj = jnp.arange(n)
    pw = jax.lax.bitcast_convert_type(
        ((127 - (jj[:, None] % 16)) << 23).astype(jnp.int32), jnp.float32)
    w2 = jnp.where(jj[:, None] // 16 == jnp.arange(ng)[None, :], pw,
                   0.0).astype(jnp.float32)

    idx = pl.pallas_call(
        functools.partial(_topk_idx_kernel, n=n, npts=npts),
        grid=(b, nb),
        in_specs=[
            pl.BlockSpec((1, c, r), lambda i, j: (i, 0, j)),
            pl.BlockSpec((1, c, n), lambda i, j: (i, 0, 0)),
            pl.BlockSpec((n, ng), lambda i, j: (0, 0)),
        ],
        out_specs=pl.BlockSpec((r, _K), lambda i, j: (i * 8 + j, 0)),
        out_shape=jax.ShapeDtypeStruct((npts, _K), jnp.int32),
        compiler_params=pltpu.CompilerParams(
            dimension_semantics=("arbitrary", "arbitrary")),
    )(x, x, w2)

    xt = jnp.transpose(x, (0, 2, 1)).reshape(npts, c)
    xtp = jnp.pad(xt, ((0, 0), (0, 128 - c)))
    pts_per_w = npts // _NW
    idx3 = idx.reshape(_NW, (pts_per_w * _K) // 128, 128)

    mesh = plsc.VectorSubcoreMesh(core_axis_name="c", subcore_axis_name="s")
    sc = pl.kernel(
        functools.partial(_sc_gather_max_kernel, pts_per_w=pts_per_w, c=c),
        out_type=jax.ShapeDtypeStruct((npts, c), jnp.float32),
        mesh=mesh,
        scratch_types=[
            pltpu.VMEM(((pts_per_w * _K) // 128, 128), jnp.int32),
            pltpu.VMEM((128, 128), jnp.float32),
            pltpu.VMEM((128, 128), jnp.float32),
            pltpu.VMEM((pts_per_w, c), jnp.float32),
            pltpu.SemaphoreType.DMA,
            pltpu.SemaphoreType.DMA,
        ],
    )
    xa = sc(xtp, idx3)

    y, maug = pl.pallas_call(
        _conv_stats_kernel,
        grid=(b, nb),
        in_specs=[
            pl.BlockSpec((r, c), lambda i, j: (i * 8 + j, 0)),
            pl.BlockSpec((c, c), lambda i, j: (0, 0)),
            pl.BlockSpec((c, 1), lambda i, j: (0, 0)),
        ],
        out_specs=[
            pl.BlockSpec((1, c, r), lambda i, j: (i, 0, j)),
            pl.BlockSpec((c + 1, c + 1), lambda i, j: (0, 0)),
        ],
        out_shape=[
            jax.ShapeDtypeStruct((b, c, n), jnp.float32),
            jax.ShapeDtypeStruct((c + 1, c + 1), jnp.float32),
        ],
        compiler_params=pltpu.CompilerParams(
            dimension_semantics=("arbitrary", "arbitrary")),
    )(xa, conv_w, bias)

    out = pl.pallas_call(
        functools.partial(_bn_kernel, cnt=float(b * n), c=c),
        grid=(b,),
        in_specs=[
            pl.BlockSpec((1, c, n), lambda i: (i, 0, 0)),
            pl.BlockSpec((c + 1, c + 1), lambda i: (0, 0)),
            pl.BlockSpec((c, 1), lambda i: (0, 0)),
            pl.BlockSpec((c, 1), lambda i: (0, 0)),
        ],
        out_specs=pl.BlockSpec((1, c, n), lambda i: (i, 0, 0)),
        out_shape=jax.ShapeDtypeStruct((b, c, n), jnp.float32),
        compiler_params=pltpu.CompilerParams(
            dimension_semantics=("arbitrary",)),
    )(y, maug, gamma, beta)
    return out


# speedup vs baseline: 2.0993x; 2.0993x over previous
import functools

import jax
import jax.numpy as jnp
from jax import lax
from jax.experimental import pallas as pl
from jax.experimental.pallas import tpu as pltpu
from jax.experimental.pallas import tpu_sc as plsc

_K = 16
_EPS = 1e-5
_R = 512

_NC = 2
_NS = 16
_NW = _NC * _NS


def _topk_idx_kernel(xr_ref, xc_ref, idx_ref, *, n, npts):
    bi = pl.program_id(0)
    xrb = xr_ref[0]
    xcb = xc_ref[0]
    g = jax.lax.dot_general(xrb, xcb, (((0,), (0,)), ((), ())),
                            preferred_element_type=jnp.float32)
    inner = -2.0 * g
    xx_r = jnp.sum(xrb * xrb, axis=0)
    xx_c = jnp.sum(xcb * xcb, axis=0)
    pd = (-xx_r[:, None] - inner) - xx_c[None, :]

    iota = jax.lax.broadcasted_iota(jnp.int32, pd.shape, 1)
    base = bi * n
    cols = []
    for _ in range(_K):
        jmin = jnp.argmax(pd, axis=1).astype(jnp.int32)
        onehot_b = iota == jmin[:, None]
        pd = jnp.where(onehot_b, -jnp.inf, pd)
        cols.append((jmin + base)[:, None])
    idx_ref[...] = jnp.concatenate(cols, axis=1)


def _sc_gather_max_kernel(xt_hbm, idx_hbm, out_hbm, idxv, rows0, rows1, acc,
                          sem0, sem1, *, pts_per_w, c):
    wid = lax.axis_index("s") * _NC + lax.axis_index("c")
    pltpu.sync_copy(idx_hbm.at[wid], idxv)
    nchunks = idxv.shape[0]
    ppc = 128 // _K
    ncc = c // 16

    def _fire(ci, rbuf, sem):
        pltpu.make_async_copy(xt_hbm.at[idxv.at[ci]], rbuf, sem).start()

    def _drain(rbuf, sem):
        pltpu.make_async_copy(xt_hbm.at[idxv.at[0]], rbuf, sem).wait()

    def _compute(ci, rbuf):
        def pt_body(p, carry):
            orow = ci * ppc + p
            for cc in range(ncc):
                m = rbuf[p * _K, pl.ds(cc * 16, 16)]
                for j in range(1, _K):
                    m = jnp.maximum(m, rbuf[p * _K + j, pl.ds(cc * 16, 16)])
                acc[orow, pl.ds(cc * 16, 16)] = m
            return carry
        lax.fori_loop(0, ppc, pt_body, 0, unroll=True)

    _fire(0, rows0, sem0)

    def body2(i2, carry):
        ci0 = i2 * 2
        _fire(ci0 + 1, rows1, sem1)
        _drain(rows0, sem0)
        _compute(ci0, rows0)

        @pl.when(i2 < nchunks // 2 - 1)
        def _():
            _fire(ci0 + 2, rows0, sem0)

        _drain(rows1, sem1)
        _compute(ci0 + 1, rows1)
        return carry

    lax.fori_loop(0, nchunks // 2, body2, 0)
    pltpu.sync_copy(acc, out_hbm.at[pl.ds(wid * pts_per_w, pts_per_w)])


def _conv_stats_kernel(xa_ref, w_ref, b_ref, y_ref, maug_ref):
    bi = pl.program_id(0)
    nb = pl.program_id(1)
    xa = xa_ref[...]
    y = jax.lax.dot_general(w_ref[...], xa, (((1,), (1,)), ((), ())),
                            preferred_element_type=jnp.float32)
    y = y + b_ref[...]
    y_ref[0] = y
    r = y.shape[1]
    aug = jnp.concatenate([y, jnp.ones((1, r), jnp.float32)], axis=0)
    contrib = jax.lax.dot_general(aug, aug, (((1,), (1,)), ((), ())),
                                  preferred_element_type=jnp.float32,
                                  precision=jax.lax.Precision.HIGHEST)

    @pl.when(jnp.logical_and(bi == 0, nb == 0))
    def _():
        maug_ref[...] = jnp.zeros_like(maug_ref)

    maug_ref[...] += contrib


def _bn_kernel(y_ref, maug_ref, gamma_ref, beta_ref, out_ref, *, cnt, c):
    maug = maug_ref[...]
    m = maug[:c, :c]
    s1 = maug[c, :c]
    eye = (jax.lax.broadcasted_iota(jnp.int32, (c, c), 0)
           == jax.lax.broadcasted_iota(jnp.int32, (c, c), 1))
    diag = jnp.sum(jnp.where(eye, m, 0.0), axis=1)
    mean = s1 / cnt
    var = diag / cnt - mean * mean
    inv = 1.0 / jnp.sqrt(var + _EPS)
    scale = gamma_ref[...][:, 0] * inv
    shift = beta_ref[...][:, 0] - mean * scale
    yb = y_ref[0]
    out_ref[0] = jnp.maximum(yb * scale[:, None] + shift[:, None], 0.0)


def kernel(x, conv_w, conv_b, bn_gamma, bn_beta):
    b, c, n = x.shape
    r = _R
    nb = n // r
    npts = b * n
    bias = conv_b.reshape(c, 1)
    gamma = bn_gamma.reshape(c, 1)
    beta = bn_beta.reshape(c, 1)

    idx = pl.pallas_call(
        functools.partial(_topk_idx_kernel, n=n, npts=npts),
        grid=(b, nb),
        in_specs=[
            pl.BlockSpec((1, c, r), lambda i, j: (i, 0, j)),
            pl.BlockSpec((1, c, n), lambda i, j: (i, 0, 0)),
        ],
        out_specs=pl.BlockSpec((r, _K), lambda i, j: (i * nb + j, 0)),
        out_shape=jax.ShapeDtypeStruct((npts, _K), jnp.int32),
        compiler_params=pltpu.CompilerParams(
            dimension_semantics=("arbitrary", "arbitrary")),
    )(x, x)

    xt = jnp.transpose(x, (0, 2, 1)).reshape(npts, c)
    xtp = jnp.pad(xt, ((0, 0), (0, 128 - c)))
    pts_per_w = npts // _NW
    idx3 = idx.reshape(_NW, (pts_per_w * _K) // 128, 128)

    mesh = plsc.VectorSubcoreMesh(core_axis_name="c", subcore_axis_name="s")
    sc = pl.kernel(
        functools.partial(_sc_gather_max_kernel, pts_per_w=pts_per_w, c=c),
        out_type=jax.ShapeDtypeStruct((npts, c), jnp.float32),
        mesh=mesh,
        scratch_types=[
            pltpu.VMEM(((pts_per_w * _K) // 128, 128), jnp.int32),
            pltpu.VMEM((128, 128), jnp.float32),
            pltpu.VMEM((128, 128), jnp.float32),
            pltpu.VMEM((pts_per_w, c), jnp.float32),
            pltpu.SemaphoreType.DMA,
            pltpu.SemaphoreType.DMA,
        ],
    )
    xa = sc(xtp, idx3)

    y, maug = pl.pallas_call(
        _conv_stats_kernel,
        grid=(b, nb),
        in_specs=[
            pl.BlockSpec((r, c), lambda i, j: (i * nb + j, 0)),
            pl.BlockSpec((c, c), lambda i, j: (0, 0)),
            pl.BlockSpec((c, 1), lambda i, j: (0, 0)),
        ],
        out_specs=[
            pl.BlockSpec((1, c, r), lambda i, j: (i, 0, j)),
            pl.BlockSpec((c + 1, c + 1), lambda i, j: (0, 0)),
        ],
        out_shape=[
            jax.ShapeDtypeStruct((b, c, n), jnp.float32),
            jax.ShapeDtypeStruct((c + 1, c + 1), jnp.float32),
        ],
        compiler_params=pltpu.CompilerParams(
            dimension_semantics=("arbitrary", "arbitrary")),
    )(xa, conv_w, bias)

    out = pl.pallas_call(
        functools.partial(_bn_kernel, cnt=float(b * n), c=c),
        grid=(b,),
        in_specs=[
            pl.BlockSpec((1, c, n), lambda i: (i, 0, 0)),
            pl.BlockSpec((c + 1, c + 1), lambda i: (0, 0)),
            pl.BlockSpec((c, 1), lambda i: (0, 0)),
            pl.BlockSpec((c, 1), lambda i: (0, 0)),
        ],
        out_specs=pl.BlockSpec((1, c, n), lambda i: (i, 0, 0)),
        out_shape=jax.ShapeDtypeStruct((b, c, n), jnp.float32),
        compiler_params=pltpu.CompilerParams(
            dimension_semantics=("arbitrary",)),
    )(y, maug, gamma, beta)
    return out


# speedup vs baseline: 2.1414x; 1.0201x over previous
import functools

import jax
import jax.numpy as jnp
from jax import lax
from jax.experimental import pallas as pl
from jax.experimental.pallas import tpu as pltpu
from jax.experimental.pallas import tpu_sc as plsc

_K = 16
_EPS = 1e-5
_R = 1024

_NC = 2
_NS = 16
_NW = _NC * _NS


def _topk_idx_kernel(xr_ref, xc_ref, idx_ref, *, n, npts):
    bi = pl.program_id(0)
    xrb = xr_ref[0]
    xcb = xc_ref[0]
    g = jax.lax.dot_general(xrb, xcb, (((0,), (0,)), ((), ())),
                            preferred_element_type=jnp.float32)
    inner = -2.0 * g
    xx_r = jnp.sum(xrb * xrb, axis=0)
    xx_c = jnp.sum(xcb * xcb, axis=0)
    pd = (-xx_r[:, None] - inner) - xx_c[None, :]

    iota = jax.lax.broadcasted_iota(jnp.int32, pd.shape, 1)
    base = bi * n
    cols = []
    for _ in range(_K):
        jmin = jnp.argmax(pd, axis=1).astype(jnp.int32)
        onehot_b = iota == jmin[:, None]
        pd = jnp.where(onehot_b, -jnp.inf, pd)
        cols.append((jmin + base)[:, None])
    idx_ref[...] = jnp.concatenate(cols, axis=1)


def _sc_gather_max_kernel(xt_hbm, idx_hbm, out_hbm, idxv, rows0, rows1, acc,
                          sem0, sem1, *, pts_per_w, c):
    wid = lax.axis_index("s") * _NC + lax.axis_index("c")
    pltpu.sync_copy(idx_hbm.at[wid], idxv)
    nchunks = idxv.shape[0]
    ppc = 128 // _K
    ncc = c // 16

    def _fire(ci, rbuf, sem):
        pltpu.make_async_copy(xt_hbm.at[idxv.at[ci]], rbuf, sem).start()

    def _drain(rbuf, sem):
        pltpu.make_async_copy(xt_hbm.at[idxv.at[0]], rbuf, sem).wait()

    def _compute(ci, rbuf):
        def pt_body(p, carry):
            orow = ci * ppc + p
            for cc in range(ncc):
                m = rbuf[p * _K, pl.ds(cc * 16, 16)]
                for j in range(1, _K):
                    m = jnp.maximum(m, rbuf[p * _K + j, pl.ds(cc * 16, 16)])
                acc[orow, pl.ds(cc * 16, 16)] = m
            return carry
        lax.fori_loop(0, ppc, pt_body, 0, unroll=True)

    _fire(0, rows0, sem0)

    def body2(i2, carry):
        ci0 = i2 * 2
        _fire(ci0 + 1, rows1, sem1)
        _drain(rows0, sem0)
        _compute(ci0, rows0)

        @pl.when(i2 < nchunks // 2 - 1)
        def _():
            _fire(ci0 + 2, rows0, sem0)

        _drain(rows1, sem1)
        _compute(ci0 + 1, rows1)
        return carry

    lax.fori_loop(0, nchunks // 2, body2, 0)
    pltpu.sync_copy(acc, out_hbm.at[pl.ds(wid * pts_per_w, pts_per_w)])


def _conv_stats_kernel(xa_ref, w_ref, b_ref, y_ref, maug_ref):
    bi = pl.program_id(0)
    nb = pl.program_id(1)
    xa = xa_ref[...]
    y = jax.lax.dot_general(w_ref[...], xa, (((1,), (1,)), ((), ())),
                            preferred_element_type=jnp.float32)
    y = y + b_ref[...]
    y_ref[0] = y
    r = y.shape[1]
    aug = jnp.concatenate([y, jnp.ones((1, r), jnp.float32)], axis=0)
    contrib = jax.lax.dot_general(aug, aug, (((1,), (1,)), ((), ())),
                                  preferred_element_type=jnp.float32,
                                  precision=jax.lax.Precision.HIGHEST)

    @pl.when(jnp.logical_and(bi == 0, nb == 0))
    def _():
        maug_ref[...] = jnp.zeros_like(maug_ref)

    maug_ref[...] += contrib


def _bn_kernel(y_ref, maug_ref, gamma_ref, beta_ref, out_ref, *, cnt, c):
    maug = maug_ref[...]
    m = maug[:c, :c]
    s1 = maug[c, :c]
    eye = (jax.lax.broadcasted_iota(jnp.int32, (c, c), 0)
           == jax.lax.broadcasted_iota(jnp.int32, (c, c), 1))
    diag = jnp.sum(jnp.where(eye, m, 0.0), axis=1)
    mean = s1 / cnt
    var = diag / cnt - mean * mean
    inv = 1.0 / jnp.sqrt(var + _EPS)
    scale = gamma_ref[...][:, 0] * inv
    shift = beta_ref[...][:, 0] - mean * scale
    yb = y_ref[0]
    out_ref[0] = jnp.maximum(yb * scale[:, None] + shift[:, None], 0.0)


def kernel(x, conv_w, conv_b, bn_gamma, bn_beta):
    b, c, n = x.shape
    r = _R
    nb = n // r
    npts = b * n
    bias = conv_b.reshape(c, 1)
    gamma = bn_gamma.reshape(c, 1)
    beta = bn_beta.reshape(c, 1)

    idx = pl.pallas_call(
        functools.partial(_topk_idx_kernel, n=n, npts=npts),
        grid=(b, nb),
        in_specs=[
            pl.BlockSpec((1, c, r), lambda i, j: (i, 0, j)),
            pl.BlockSpec((1, c, n), lambda i, j: (i, 0, 0)),
        ],
        out_specs=pl.BlockSpec((r, _K), lambda i, j: (i * nb + j, 0)),
        out_shape=jax.ShapeDtypeStruct((npts, _K), jnp.int32),
        compiler_params=pltpu.CompilerParams(
            dimension_semantics=("arbitrary", "arbitrary")),
    )(x, x)

    xt = jnp.transpose(x, (0, 2, 1)).reshape(npts, c)
    xtp = jnp.pad(xt, ((0, 0), (0, 128 - c)))
    pts_per_w = npts // _NW
    idx3 = idx.reshape(_NW, (pts_per_w * _K) // 128, 128)

    mesh = plsc.VectorSubcoreMesh(core_axis_name="c", subcore_axis_name="s")
    sc = pl.kernel(
        functools.partial(_sc_gather_max_kernel, pts_per_w=pts_per_w, c=c),
        out_type=jax.ShapeDtypeStruct((npts, c), jnp.float32),
        mesh=mesh,
        scratch_types=[
            pltpu.VMEM(((pts_per_w * _K) // 128, 128), jnp.int32),
            pltpu.VMEM((128, 128), jnp.float32),
            pltpu.VMEM((128, 128), jnp.float32),
            pltpu.VMEM((pts_per_w, c), jnp.float32),
            pltpu.SemaphoreType.DMA,
            pltpu.SemaphoreType.DMA,
        ],
    )
    xa = sc(xtp, idx3)

    y, maug = pl.pallas_call(
        _conv_stats_kernel,
        grid=(b, nb),
        in_specs=[
            pl.BlockSpec((r, c), lambda i, j: (i * nb + j, 0)),
            pl.BlockSpec((c, c), lambda i, j: (0, 0)),
            pl.BlockSpec((c, 1), lambda i, j: (0, 0)),
        ],
        out_specs=[
            pl.BlockSpec((1, c, r), lambda i, j: (i, 0, j)),
            pl.BlockSpec((c + 1, c + 1), lambda i, j: (0, 0)),
        ],
        out_shape=[
            jax.ShapeDtypeStruct((b, c, n), jnp.float32),
            jax.ShapeDtypeStruct((c + 1, c + 1), jnp.float32),
        ],
        compiler_params=pltpu.CompilerParams(
            dimension_semantics=("arbitrary", "arbitrary")),
    )(xa, conv_w, bias)

    out = pl.pallas_call(
        functools.partial(_bn_kernel, cnt=float(b * n), c=c),
        grid=(b,),
        in_specs=[
            pl.BlockSpec((1, c, n), lambda i: (i, 0, 0)),
            pl.BlockSpec((c + 1, c + 1), lambda i: (0, 0)),
            pl.BlockSpec((c, 1), lambda i: (0, 0)),
            pl.BlockSpec((c, 1), lambda i: (0, 0)),
        ],
        out_specs=pl.BlockSpec((1, c, n), lambda i: (i, 0, 0)),
        out_shape=jax.ShapeDtypeStruct((b, c, n), jnp.float32),
        compiler_params=pltpu.CompilerParams(
            dimension_semantics=("arbitrary",)),
    )(y, maug, gamma, beta)
    return out
